# speedup vs baseline: 1.7973x; 1.7973x over previous
import functools

import jax
import jax.numpy as jnp
from jax import lax
from jax.experimental import pallas as pl
from jax.experimental.pallas import tpu as pltpu
from jax.experimental.pallas import tpu_sc as plsc


def _fused_table_body(e_ref, w_ref, b_ref, m_ref):
    m_ref[...] = (
        jnp.dot(e_ref[...], w_ref[...].T, preferred_element_type=jnp.float32)
        + b_ref[...]
    )


def _make_fused_table(V, H):
    return pl.pallas_call(
        _fused_table_body,
        out_shape=jax.ShapeDtypeStruct((V, V), jnp.float32),
    )


def _make_gather(V, D, B):
    info = plsc.get_sparse_core_info()
    NW = info.num_cores * info.num_subcores
    b_per_w = B // NW
    CHUNK = 128
    n_chunks = b_per_w // CHUNK
    mesh = plsc.VectorSubcoreMesh(core_axis_name="c", subcore_axis_name="s")

    @functools.partial(
        pl.kernel,
        mesh=mesh,
        out_type=jax.ShapeDtypeStruct((B, D), jnp.float32),
        scratch_types=[
            pltpu.VMEM((CHUNK,), jnp.int32),
            pltpu.VMEM((CHUNK, D), jnp.float32),
            pltpu.SemaphoreType.DMA,
        ],
    )
    def gather_kernel(table_hbm, idx_hbm, out_hbm, idx_v, rows_v, sem):
        wid = lax.axis_index("s") * info.num_cores + lax.axis_index("c")
        base = wid * b_per_w
        for j in range(n_chunks):
            off = base + j * CHUNK
            pltpu.sync_copy(idx_hbm.at[pl.ds(off, CHUNK)], idx_v)
            pltpu.async_copy(table_hbm.at[idx_v], rows_v, sem).wait()
            pltpu.sync_copy(rows_v, out_hbm.at[pl.ds(off, CHUNK)])

    return gather_kernel


def kernel(input_ids, attention_mask, embed_weight, lm_head_weight, lm_head_bias):
    V, H = embed_weight.shape
    Bb, S = input_ids.shape
    table = _make_fused_table(V, H)(
        embed_weight, lm_head_weight, lm_head_bias.reshape(1, V)
    )
    ids = input_ids.reshape(-1)
    out = _make_gather(V, V, Bb * S)(table, ids)
    return out.reshape(Bb, S, V)


# speedup vs baseline: 1.7978x; 1.0003x over previous
import functools

import jax
import jax.numpy as jnp
from jax import lax
from jax.experimental import pallas as pl
from jax.experimental.pallas import tpu as pltpu
from jax.experimental.pallas import tpu_sc as plsc


def _fused_table_body(e_ref, w_ref, b_ref, m_ref):
    m_ref[...] = (
        jnp.dot(e_ref[...], w_ref[...].T, preferred_element_type=jnp.float32)
        + b_ref[...]
    )


def _make_fused_table(V, H):
    return pl.pallas_call(
        _fused_table_body,
        out_shape=jax.ShapeDtypeStruct((V, V), jnp.float32),
    )


def _make_gather(V, D, B, CHUNK=128):
    info = plsc.get_sparse_core_info()
    NW = info.num_cores * info.num_subcores
    b_per_w = B // NW
    n_chunks = b_per_w // CHUNK
    mesh = plsc.VectorSubcoreMesh(core_axis_name="c", subcore_axis_name="s")

    @functools.partial(
        pl.kernel,
        mesh=mesh,
        out_type=jax.ShapeDtypeStruct((B, D), jnp.float32),
        scratch_types=[
            pltpu.VMEM((n_chunks, CHUNK), jnp.int32),
            pltpu.VMEM((CHUNK, D), jnp.float32),
            pltpu.VMEM((CHUNK, D), jnp.float32),
            pltpu.SemaphoreType.DMA,
            pltpu.SemaphoreType.DMA,
        ],
    )
    def gather_kernel(table_hbm, idx_hbm, out_hbm, idx_v, rows0, rows1, sem0, sem1):
        wid = lax.axis_index("s") * info.num_cores + lax.axis_index("c")
        base = wid * b_per_w
        pltpu.sync_copy(idx_hbm.at[pl.ds(wid * n_chunks, n_chunks)], idx_v)
        rows = [rows0, rows1]
        sems = [sem0, sem1]
        cps = [None, None]
        cps[0] = pltpu.async_copy(table_hbm.at[idx_v.at[0]], rows[0], sems[0])
        for j in range(n_chunks):
            b = j & 1
            if j + 1 < n_chunks:
                cps[b ^ 1] = pltpu.async_copy(
                    table_hbm.at[idx_v.at[j + 1]], rows[b ^ 1], sems[b ^ 1]
                )
            cps[b].wait()
            pltpu.sync_copy(rows[b], out_hbm.at[pl.ds(base + j * CHUNK, CHUNK)])

    return gather_kernel


def kernel(input_ids, attention_mask, embed_weight, lm_head_weight, lm_head_bias):
    V, H = embed_weight.shape
    Bb, S = input_ids.shape
    table = _make_fused_table(V, H)(
        embed_weight, lm_head_weight, lm_head_bias.reshape(1, V)
    )
    ids = input_ids.reshape(-1, 128)
    out = _make_gather(V, V, Bb * S)(table, ids)
    return out.reshape(Bb, S, V)
